# Initial kernel scaffold; baseline (speedup 1.0000x reference)
import functools

import jax
import jax.numpy as jnp
from jax import lax
from jax.experimental import pallas as pl
from jax.experimental.pallas import tpu as pltpu
from jax.experimental.pallas import tpu_sc as plsc

N_USERS = 25000
N_ITEMS = 25000
N = N_USERERS = N_USERS + N_ITEMS
N = N_USERS + N_ITEMS
D = 64
E = 800000
LAYERS = 4

NC, NS, L = 2, 16, 16
HALF = 25088
NPAD = 2 * HALF
ROWS_T = HALF // NS
GARB = 16
ACC_ROWS = HALF + GARB
ZROWS_T = ACC_ROWS // NS
EC = 128
ES = 50176
EPAD = ES * NS
NCH_E = ES // EC
KQ = 4
RC = 112
NCH_R = ROWS_T // RC

_MESH = plsc.VectorSubcoreMesh(
    core_axis_name="c", subcore_axis_name="s", num_cores=NC, num_subcores=NS)


def _rsqrt_newton(x):
    i = plsc.bitcast(x, jnp.int32)
    y = plsc.bitcast(jnp.int32(0x5F3759DF) - (i >> 1), jnp.float32)
    for _ in range(3):
        y = y * (1.5 - 0.5 * x * y * y)
    return y


@functools.partial(
    pl.kernel,
    out_type=(
        jax.ShapeDtypeStruct((NPAD, D), jnp.float32),
        jax.ShapeDtypeStruct((NPAD,), jnp.float32),
    ),
    mesh=_MESH,
    scratch_types=(
        pltpu.MemorySpace.VMEM_SHARED((ACC_ROWS, L), jnp.float32),
        pltpu.VMEM((EC, L), jnp.float32),
        pltpu.VMEM((EC, L), jnp.float32),
        pltpu.VMEM((EC,), jnp.int32),
        pltpu.VMEM((EC,), jnp.int32),
        pltpu.VMEM((RC, L), jnp.float32),
        pltpu.VMEM((RC,), jnp.float32),
        pltpu.VMEM((RC, D), jnp.float32),
        pltpu.VMEM((RC, D), jnp.float32),
    ),
)
def _deg_kernel(dst_hbm, emb0_hbm, z0_hbm, dinv_hbm,
                acc16, ones_v, zb16, dstv, idxv, cbuf, dv, ebuf, zbuf):
    c = lax.axis_index("c")
    s = lax.axis_index("s")

    @pl.loop(0, EC)
    def _fill(i):
        ones_v[i, :] = jnp.ones((L,), jnp.float32)
        zb16[i, :] = jnp.zeros((L,), jnp.float32)

    z0r = s * ZROWS_T

    @pl.loop(0, ZROWS_T // EC)
    def _zero(i):
        pltpu.sync_copy(zb16, acc16.at[pl.ds(z0r + i * EC, EC)])

    _ztail = ZROWS_T - (ZROWS_T // EC) * EC
    pltpu.sync_copy(zb16.at[pl.ds(0, _ztail)],
                    acc16.at[pl.ds(z0r + (ZROWS_T // EC) * EC, _ztail)])
    plsc.subcore_barrier()

    lo = c * HALF
    garb = HALF + s

    @pl.loop(0, NCH_E)
    def _sweep(g):
        base = s * ES + g * EC
        pltpu.sync_copy(dst_hbm.at[pl.ds(base, EC)], dstv)
        for j in range(EC // L):
            d = dstv[pl.ds(j * L, L)]
            inh = (d >= lo) & (d < lo + HALF)
            idxv[pl.ds(j * L, L)] = jnp.where(inh, d - lo, garb)
        pltpu.sync_copy(ones_v, acc16.at[idxv], add=True)

    plsc.subcore_barrier()

    rbase = s * ROWS_T
    gbase = c * HALF + s * ROWS_T

    @pl.loop(0, NCH_R)
    def _out(k):
        g0 = gbase + k * RC
        pltpu.sync_copy(acc16.at[pl.ds(rbase + k * RC, RC)], cbuf)
        pltpu.sync_copy(emb0_hbm.at[pl.ds(g0, RC)], ebuf)
        for q in range(RC // L):
            ridx = lax.iota(jnp.int32, L) + q * L
            deg = plsc.load_gather(cbuf, [ridx, jnp.zeros((L,), jnp.int32)])
            dinv16 = jnp.where(deg > 0.0, _rsqrt_newton(deg), 0.0)
            dv[pl.ds(q * L, L)] = dinv16

        @pl.loop(0, RC)
        def _scale(r):
            sp = plsc.load_gather(dv, [jnp.full((L,), r, jnp.int32)])
            for j in range(D // L):
                zbuf[r, pl.ds(j * L, L)] = ebuf[r, pl.ds(j * L, L)] * sp

        pltpu.sync_copy(zbuf, z0_hbm.at[pl.ds(g0, RC)])
        pltpu.sync_copy(dv, dinv_hbm.at[pl.ds(g0, RC)])


@functools.partial(
    pl.kernel,
    out_type=(
        jax.ShapeDtypeStruct((NPAD, D), jnp.float32),
        jax.ShapeDtypeStruct((NPAD, D), jnp.float32),
    ),
    mesh=_MESH,
    scratch_types=(
        pltpu.MemorySpace.VMEM_SHARED((ACC_ROWS, D), jnp.float32),
        pltpu.VMEM((EC, D), jnp.float32),
        pltpu.VMEM((KQ, EC), jnp.int32),
        pltpu.VMEM((EC,), jnp.int32),
        pltpu.VMEM((KQ, EC), jnp.int32),
        pltpu.VMEM((KQ, EC, D), jnp.float32),
        pltpu.VMEM((RC, D), jnp.float32),
        pltpu.VMEM((RC, D), jnp.float32),
        pltpu.VMEM((RC,), jnp.float32),
        pltpu.SemaphoreType.DMA,
    ),
)
def _layer_kernel(src_hbm, dst_hbm, z_hbm, dinv_hbm, tot_hbm,
                  z_out, tot_out,
                  acc, zb, srcv, dstv, idxv, rows, abuf, tbuf, dv, sem):
    c = lax.axis_index("c")
    s = lax.axis_index("s")

    @pl.loop(0, EC)
    def _fill(i):
        for j in range(D // L):
            zb[i, pl.ds(j * L, L)] = jnp.zeros((L,), jnp.float32)

    z0r = s * ZROWS_T

    @pl.loop(0, ZROWS_T // EC)
    def _zero(i):
        pltpu.sync_copy(zb, acc.at[pl.ds(z0r + i * EC, EC)])

    _ztail = ZROWS_T - (ZROWS_T // EC) * EC
    pltpu.sync_copy(zb.at[pl.ds(0, _ztail)],
                    acc.at[pl.ds(z0r + (ZROWS_T // EC) * EC, _ztail)])
    plsc.subcore_barrier()

    lo = c * HALF
    garb = HALF + s

    @pl.loop(0, NCH_E // KQ)
    def _sweep(t):
        cps = []
        for b in range(KQ):
            base = s * ES + (t * KQ + b) * EC
            pltpu.sync_copy(src_hbm.at[pl.ds(base, EC)], srcv.at[b])
            pltpu.sync_copy(dst_hbm.at[pl.ds(base, EC)], dstv)
            for j in range(EC // L):
                d = dstv[pl.ds(j * L, L)]
                inh = (d >= lo) & (d < lo + HALF)
                idxv[b, pl.ds(j * L, L)] = jnp.where(inh, d - lo, garb)
            cps.append(pltpu.async_copy(z_hbm.at[srcv.at[b]], rows.at[b], sem))
        for cp in cps:
            cp.wait()
        for b in range(KQ):
            pltpu.sync_copy(rows.at[b], acc.at[idxv.at[b]], add=True)

    plsc.subcore_barrier()

    rbase = s * ROWS_T
    gbase = c * HALF + s * ROWS_T

    @pl.loop(0, NCH_R)
    def _out(k):
        g0 = gbase + k * RC
        pltpu.sync_copy(acc.at[pl.ds(rbase + k * RC, RC)], abuf)
        pltpu.sync_copy(tot_hbm.at[pl.ds(g0, RC)], tbuf)
        pltpu.sync_copy(dinv_hbm.at[pl.ds(g0, RC)], dv)

        @pl.loop(0, RC)
        def _scale(r):
            sp = plsc.load_gather(dv, [jnp.full((L,), r, jnp.int32)])
            for j in range(D // L):
                a = abuf[r, pl.ds(j * L, L)] * sp
                tbuf[r, pl.ds(j * L, L)] = tbuf[r, pl.ds(j * L, L)] + a
                abuf[r, pl.ds(j * L, L)] = a * sp

        pltpu.sync_copy(tbuf, tot_out.at[pl.ds(g0, RC)])
        pltpu.sync_copy(abuf, z_out.at[pl.ds(g0, RC)])


def kernel(edge_index, user_weight, item_weight):
    src = edge_index[0].astype(jnp.int32)
    dst = edge_index[1].astype(jnp.int32)
    pe = EPAD - E
    pad_src = (jnp.arange(pe, dtype=jnp.int32) * 997) % N
    pad_dst = jnp.full((pe,), NPAD, dtype=jnp.int32)
    srcp = jnp.concatenate([src, pad_src])
    dstp = jnp.concatenate([dst, pad_dst])

    emb0 = jnp.concatenate([user_weight, item_weight], axis=0)
    emb0p = jnp.pad(emb0, ((0, NPAD - N), (0, 0)))

    z, dinv = _deg_kernel(dstp, emb0p)
    tot = emb0p
    for _ in range(LAYERS):
        z, tot = _layer_kernel(srcp, dstp, z, dinv, tot)

    out = tot * (1.0 / ((LAYERS + 1) * (LAYERS + 1)))
    return out[:N_USERS], out[N_USERS:N]


# speedup vs baseline: 6.8764x; 6.8764x over previous
import functools

import jax
import jax.numpy as jnp
from jax import lax
from jax.experimental import pallas as pl
from jax.experimental.pallas import tpu as pltpu
from jax.experimental.pallas import tpu_sc as plsc

N_USERS = 25000
N_ITEMS = 25000
N = N_USERS + N_ITEMS
D = 64
E = 800000
LAYERS = 4

NC, NS, L = 2, 16, 16
HALF = 25088
NPAD = 2 * HALF
ROWS_T = HALF // NS
GARB = 16
ACC_ROWS = HALF + GARB
ZROWS_T = ACC_ROWS // NS
EC = 128
ES = 50176
EPAD = ES * NS
NCH_E = ES // EC
KQ = 2
ZB = 32
BLK = 512


def _deg_body(dst_hbm, deg16_hbm, acc16, ones_v, zb16, dstv, idxv):
    c = lax.axis_index("c")
    s = lax.axis_index("s")

    @pl.loop(0, EC)
    def _fill(i):
        ones_v[i, :] = jnp.ones((L,), jnp.float32)
        zb16[i, :] = jnp.zeros((L,), jnp.float32)

    z0r = s * ZROWS_T

    @pl.loop(0, ZROWS_T // EC)
    def _zero(i):
        pltpu.sync_copy(zb16, acc16.at[pl.ds(z0r + i * EC, EC)])

    _ztail = ZROWS_T - (ZROWS_T // EC) * EC
    pltpu.sync_copy(zb16.at[pl.ds(0, _ztail)],
                    acc16.at[pl.ds(z0r + (ZROWS_T // EC) * EC, _ztail)])
    plsc.subcore_barrier()

    lo = c * HALF
    garb = HALF + s

    @pl.loop(0, NCH_E)
    def _sweep(g):
        base = s * ES + g * EC
        pltpu.sync_copy(dst_hbm.at[pl.ds(base, EC)], dstv)
        for j in range(EC // L):
            d = dstv[pl.ds(j * L, L)]
            inh = (d >= lo) & (d < lo + HALF)
            idxv[pl.ds(j * L, L)] = jnp.where(inh, d - lo, garb)
        pltpu.sync_copy(ones_v, acc16.at[idxv], add=True)

    plsc.subcore_barrier()

    rbase = s * ROWS_T
    gbase = c * HALF + s * ROWS_T
    pltpu.sync_copy(acc16.at[pl.ds(rbase, ROWS_T)],
                    deg16_hbm.at[pl.ds(gbase, ROWS_T)])


def _layer_body(src_hbm, dst_hbm, z_hbm, acc_hbm,
                acc, zb, srcv, dstv, idxv, rows, sem):
    c = lax.axis_index("c")
    s = lax.axis_index("s")

    @pl.loop(0, ZB)
    def _fill(i):
        for j in range(D // L):
            zb[i, pl.ds(j * L, L)] = jnp.zeros((L,), jnp.float32)

    z0r = s * ZROWS_T

    @pl.loop(0, ZROWS_T // ZB)
    def _zero(i):
        pltpu.sync_copy(zb, acc.at[pl.ds(z0r + i * ZB, ZB)])

    _ztail = ZROWS_T - (ZROWS_T // ZB) * ZB
    pltpu.sync_copy(zb.at[pl.ds(0, _ztail)],
                    acc.at[pl.ds(z0r + (ZROWS_T // ZB) * ZB, _ztail)])
    plsc.subcore_barrier()

    lo = c * HALF
    garb = HALF + s

    @pl.loop(0, NCH_E // KQ)
    def _sweep(t):
        cps = []
        for b in range(KQ):
            base = s * ES + (t * KQ + b) * EC
            pltpu.sync_copy(src_hbm.at[pl.ds(base, EC)], srcv.at[b])
            pltpu.sync_copy(dst_hbm.at[pl.ds(base, EC)], dstv)
            for j in range(EC // L):
                d = dstv[pl.ds(j * L, L)]
                inh = (d >= lo) & (d < lo + HALF)
                idxv[b, pl.ds(j * L, L)] = jnp.where(inh, d - lo, garb)
            cps.append(pltpu.async_copy(z_hbm.at[srcv.at[b]], rows.at[b], sem))
        for cp in cps:
            cp.wait()
        for b in range(KQ):
            pltpu.sync_copy(rows.at[b], acc.at[idxv.at[b]], add=True)

    plsc.subcore_barrier()

    rbase = s * ROWS_T
    gbase = c * HALF + s * ROWS_T
    pltpu.sync_copy(acc.at[pl.ds(rbase, ROWS_T)],
                    acc_hbm.at[pl.ds(gbase, ROWS_T)])


def _dinv_tc_body(deg16_ref, emb0_ref, z0_ref, dinvrow_ref):
    deg = deg16_ref[:, 0:1]
    dinv = jnp.where(deg > 0.0, lax.rsqrt(jnp.maximum(deg, 1e-12)), 0.0)
    dinvrow_ref[...] = jnp.broadcast_to(dinv, (BLK, D))
    z0_ref[...] = emb0_ref[...] * dinv


def _scale_tc_body(dinvrow_ref, acc_ref, tot_ref, z_ref, totout_ref):
    dr = dinvrow_ref[...]
    emb = acc_ref[...] * dr
    totout_ref[...] = tot_ref[...] + emb
    z_ref[...] = emb * dr


@functools.lru_cache(maxsize=None)
def _build_kernels():
    mesh = plsc.VectorSubcoreMesh(
        core_axis_name="c", subcore_axis_name="s",
        num_cores=NC, num_subcores=NS)
    deg_k = pl.kernel(
        _deg_body,
        out_type=(jax.ShapeDtypeStruct((NPAD, L), jnp.float32),),
        mesh=mesh,
        compiler_params=pltpu.CompilerParams(use_tc_tiling_on_sc=False),
        scratch_types=(
            pltpu.MemorySpace.VMEM_SHARED((ACC_ROWS, L), jnp.float32),
            pltpu.VMEM((EC, L), jnp.float32),
            pltpu.VMEM((EC, L), jnp.float32),
            pltpu.VMEM((EC,), jnp.int32),
            pltpu.VMEM((EC,), jnp.int32),
        ),
    )
    layer_k = pl.kernel(
        _layer_body,
        out_type=(jax.ShapeDtypeStruct((NPAD, D), jnp.float32),),
        mesh=mesh,
        compiler_params=pltpu.CompilerParams(use_tc_tiling_on_sc=False),
        scratch_types=(
            pltpu.MemorySpace.VMEM_SHARED((ACC_ROWS, D), jnp.float32),
            pltpu.VMEM((ZB, D), jnp.float32),
            pltpu.VMEM((KQ, EC), jnp.int32),
            pltpu.VMEM((EC,), jnp.int32),
            pltpu.VMEM((KQ, EC), jnp.int32),
            pltpu.VMEM((KQ, EC, D), jnp.float32),
            pltpu.SemaphoreType.DMA,
        ),
    )
    grid = (NPAD // BLK,)
    blk2 = pl.BlockSpec((BLK, D), lambda i: (i, 0))
    blk16 = pl.BlockSpec((BLK, L), lambda i: (i, 0))
    dinv_k = pl.pallas_call(
        _dinv_tc_body,
        grid=grid,
        in_specs=[blk16, blk2],
        out_specs=[blk2, blk2],
        out_shape=(
            jax.ShapeDtypeStruct((NPAD, D), jnp.float32),
            jax.ShapeDtypeStruct((NPAD, D), jnp.float32),
        ),
    )
    scale_k = pl.pallas_call(
        _scale_tc_body,
        grid=grid,
        in_specs=[blk2, blk2, blk2],
        out_specs=[blk2, blk2],
        out_shape=(
            jax.ShapeDtypeStruct((NPAD, D), jnp.float32),
            jax.ShapeDtypeStruct((NPAD, D), jnp.float32),
        ),
    )
    return deg_k, layer_k, dinv_k, scale_k


def kernel(edge_index, user_weight, item_weight):
    src = edge_index[0].astype(jnp.int32)
    dst = edge_index[1].astype(jnp.int32)
    pe = EPAD - E
    pad_src = (jnp.arange(pe, dtype=jnp.int32) * 997) % N
    pad_dst = jnp.full((pe,), NPAD, dtype=jnp.int32)
    srcp = jnp.concatenate([src, pad_src])
    dstp = jnp.concatenate([dst, pad_dst])

    emb0 = jnp.concatenate([user_weight, item_weight], axis=0)
    emb0p = jnp.pad(emb0, ((0, NPAD - N), (0, 0)))

    deg_k, layer_k, dinv_k, scale_k = _build_kernels()
    (deg16,) = deg_k(dstp)
    z, dinvrow = dinv_k(deg16, emb0p)
    tot = emb0p
    for _ in range(LAYERS):
        (acc,) = layer_k(srcp, dstp, z)
        z, tot = scale_k(dinvrow, acc, tot)

    out = tot * (1.0 / ((LAYERS + 1) * (LAYERS + 1)))
    return out[:N_USERS], out[N_USERS:N]


# speedup vs baseline: 11.7138x; 1.7035x over previous
import functools

import jax
import jax.numpy as jnp
from jax import lax
from jax.experimental import pallas as pl
from jax.experimental.pallas import tpu as pltpu
from jax.experimental.pallas import tpu_sc as plsc

N_USERS = 25000
N_ITEMS = 25000
N = N_USERS + N_ITEMS
D = 64
E = 800000
LAYERS = 4

NC, NS, L = 2, 16, 16
HALF = 25088
NPAD = 2 * HALF
ROWS_T = HALF // NS
GARB = 16
ACC_ROWS = HALF + GARB
ZROWS_T = ACC_ROWS // NS
EC = 128
ES = 50176
EPAD = ES * NS
NCH_E = ES // EC
BC = 8
NB = NCH_E // BC
NCH_ALL = EPAD // EC
KQ = 2
ZB = 16
BLK = 512


def _deg_body(dst_hbm, deg16_hbm, idxl_hbm, acc16, ones_v, zb16, dstb, idxb):
    c = lax.axis_index("c")
    s = lax.axis_index("s")

    @pl.loop(0, EC)
    def _fill(i):
        ones_v[i, :] = jnp.ones((L,), jnp.float32)

    @pl.loop(0, ZB)
    def _fillz(i):
        zb16[i, :] = jnp.zeros((L,), jnp.float32)

    z0r = s * ZROWS_T

    @pl.loop(0, ZROWS_T // ZB)
    def _zero(i):
        pltpu.sync_copy(zb16, acc16.at[pl.ds(z0r + i * ZB, ZB)])

    _ztail = ZROWS_T - (ZROWS_T // ZB) * ZB
    pltpu.sync_copy(zb16.at[pl.ds(0, _ztail)],
                    acc16.at[pl.ds(z0r + (ZROWS_T // ZB) * ZB, _ztail)])
    plsc.subcore_barrier()

    lo = c * HALF
    garb = HALF + s

    @pl.loop(0, NB)
    def _sweep(t):
        cbase = s * NCH_E + t * BC
        pltpu.sync_copy(dst_hbm.at[pl.ds(cbase, BC)], dstb)
        for k in range(BC):
            for j in range(EC // L):
                d = dstb[k, pl.ds(j * L, L)]
                inh = (d >= lo) & (d < lo + HALF)
                idxb[k, pl.ds(j * L, L)] = jnp.where(inh, d - lo, garb)
            pltpu.sync_copy(ones_v, acc16.at[idxb.at[k]], add=True)
        pltpu.sync_copy(idxb, idxl_hbm.at[c, pl.ds(cbase, BC)])

    plsc.subcore_barrier()

    rbase = s * ROWS_T
    gbase = c * HALF + s * ROWS_T
    pltpu.sync_copy(acc16.at[pl.ds(rbase, ROWS_T)],
                    deg16_hbm.at[pl.ds(gbase, ROWS_T)])


def _layer_body(src_hbm, idxl_hbm, z_hbm, acc_hbm,
                acc, zb, srcb, idxb, rows, sem):
    c = lax.axis_index("c")
    s = lax.axis_index("s")

    @pl.loop(0, ZB)
    def _fill(i):
        for j in range(D // L):
            zb[i, pl.ds(j * L, L)] = jnp.zeros((L,), jnp.float32)

    z0r = s * ZROWS_T

    @pl.loop(0, ZROWS_T // ZB)
    def _zero(i):
        pltpu.sync_copy(zb, acc.at[pl.ds(z0r + i * ZB, ZB)])

    _ztail = ZROWS_T - (ZROWS_T // ZB) * ZB
    pltpu.sync_copy(zb.at[pl.ds(0, _ztail)],
                    acc.at[pl.ds(z0r + (ZROWS_T // ZB) * ZB, _ztail)])
    plsc.subcore_barrier()

    @pl.loop(0, NB)
    def _sweep(t):
        cbase = s * NCH_E + t * BC
        pltpu.sync_copy(src_hbm.at[pl.ds(cbase, BC)], srcb)
        pltpu.sync_copy(idxl_hbm.at[c, pl.ds(cbase, BC)], idxb)
        cps = [None] * BC
        for k in range(KQ):
            cps[k] = pltpu.async_copy(
                z_hbm.at[srcb.at[k]], rows.at[k % KQ], sem)
        for k in range(BC):
            cps[k].wait()
            pltpu.sync_copy(rows.at[k % KQ], acc.at[idxb.at[k]], add=True)
            nk = k + KQ
            if nk < BC:
                cps[nk] = pltpu.async_copy(
                    z_hbm.at[srcb.at[nk]], rows.at[nk % KQ], sem)

    plsc.subcore_barrier()

    rbase = s * ROWS_T
    gbase = c * HALF + s * ROWS_T
    pltpu.sync_copy(acc.at[pl.ds(rbase, ROWS_T)],
                    acc_hbm.at[pl.ds(gbase, ROWS_T)])


def _dinv_tc_body(deg16_ref, emb0_ref, z0_ref, dinvrow_ref):
    deg = deg16_ref[:, 0:1]
    dinv = jnp.where(deg > 0.0, lax.rsqrt(jnp.maximum(deg, 1e-12)), 0.0)
    dinvrow_ref[...] = jnp.broadcast_to(dinv, (BLK, D))
    z0_ref[...] = emb0_ref[...] * dinv


def _scale_tc_body(dinvrow_ref, acc_ref, tot_ref, z_ref, totout_ref):
    dr = dinvrow_ref[...]
    emb = acc_ref[...] * dr
    totout_ref[...] = tot_ref[...] + emb
    z_ref[...] = emb * dr


@functools.lru_cache(maxsize=None)
def _build_kernels():
    mesh = plsc.VectorSubcoreMesh(
        core_axis_name="c", subcore_axis_name="s",
        num_cores=NC, num_subcores=NS)
    deg_k = pl.kernel(
        _deg_body,
        out_type=(
            jax.ShapeDtypeStruct((NPAD, L), jnp.float32),
            jax.ShapeDtypeStruct((NC, NCH_ALL, EC), jnp.int32),
        ),
        mesh=mesh,
        compiler_params=pltpu.CompilerParams(use_tc_tiling_on_sc=False),
        scratch_types=(
            pltpu.MemorySpace.VMEM_SHARED((ACC_ROWS, L), jnp.float32),
            pltpu.VMEM((EC, L), jnp.float32),
            pltpu.VMEM((ZB, L), jnp.float32),
            pltpu.VMEM((BC, EC), jnp.int32),
            pltpu.VMEM((BC, EC), jnp.int32),
        ),
    )
    layer_k = pl.kernel(
        _layer_body,
        out_type=(jax.ShapeDtypeStruct((NPAD, D), jnp.float32),),
        mesh=mesh,
        compiler_params=pltpu.CompilerParams(use_tc_tiling_on_sc=False),
        scratch_types=(
            pltpu.MemorySpace.VMEM_SHARED((ACC_ROWS, D), jnp.float32),
            pltpu.VMEM((ZB, D), jnp.float32),
            pltpu.VMEM((BC, EC), jnp.int32),
            pltpu.VMEM((BC, EC), jnp.int32),
            pltpu.VMEM((KQ, EC, D), jnp.float32),
            pltpu.SemaphoreType.DMA,
        ),
    )
    grid = (NPAD // BLK,)
    blk2 = pl.BlockSpec((BLK, D), lambda i: (i, 0))
    blk16 = pl.BlockSpec((BLK, L), lambda i: (i, 0))
    dinv_k = pl.pallas_call(
        _dinv_tc_body,
        grid=grid,
        in_specs=[blk16, blk2],
        out_specs=[blk2, blk2],
        out_shape=(
            jax.ShapeDtypeStruct((NPAD, D), jnp.float32),
            jax.ShapeDtypeStruct((NPAD, D), jnp.float32),
        ),
    )
    scale_k = pl.pallas_call(
        _scale_tc_body,
        grid=grid,
        in_specs=[blk2, blk2, blk2],
        out_specs=[blk2, blk2],
        out_shape=(
            jax.ShapeDtypeStruct((NPAD, D), jnp.float32),
            jax.ShapeDtypeStruct((NPAD, D), jnp.float32),
        ),
    )
    return deg_k, layer_k, dinv_k, scale_k


def kernel(edge_index, user_weight, item_weight):
    src = edge_index[0].astype(jnp.int32)
    dst = edge_index[1].astype(jnp.int32)
    pe = EPAD - E
    pad_src = (jnp.arange(pe, dtype=jnp.int32) * 997) % N
    pad_dst = jnp.full((pe,), NPAD, dtype=jnp.int32)
    srcp = jnp.concatenate([src, pad_src]).reshape(NCH_ALL, EC)
    dstp = jnp.concatenate([dst, pad_dst]).reshape(NCH_ALL, EC)

    emb0 = jnp.concatenate([user_weight, item_weight], axis=0)
    emb0p = jnp.pad(emb0, ((0, NPAD - N), (0, 0)))

    deg_k, layer_k, dinv_k, scale_k = _build_kernels()
    deg16, idxl = deg_k(dstp)
    z, dinvrow = dinv_k(deg16, emb0p)
    tot = emb0p
    for _ in range(LAYERS):
        (acc,) = layer_k(srcp, idxl, z)
        z, tot = scale_k(dinvrow, acc, tot)

    out = tot * (1.0 / ((LAYERS + 1) * (LAYERS + 1)))
    return out[:N_USERS], out[N_USERS:N]


# speedup vs baseline: 12.5177x; 1.0686x over previous
import functools

import jax
import jax.numpy as jnp
from jax import lax
from jax.experimental import pallas as pl
from jax.experimental.pallas import tpu as pltpu
from jax.experimental.pallas import tpu_sc as plsc

N_USERS = 25000
N_ITEMS = 25000
N = N_USERS + N_ITEMS
D = 64
E = 800000
LAYERS = 4

NC, NS, L = 2, 16, 16
HALF = 25088
NPAD = 2 * HALF
ROWS_T = HALF // NS
GARB = 16
ACC_ROWS = HALF + GARB
ZROWS_T = ACC_ROWS // NS
EC = 128
ES = 50176
EPAD = ES * NS
NCH_E = ES // EC
BC = 8
NB = NCH_E // BC
NCH_ALL = EPAD // EC
KQ = 2
ZB = 16
RC = 56
BLK = 512


def _deg_body(dst_hbm, deg16_hbm, idxl_hbm, acc16, ones_v, zb16, dstb, idxb):
    c = lax.axis_index("c")
    s = lax.axis_index("s")

    @pl.loop(0, EC)
    def _fill(i):
        ones_v[i, :] = jnp.ones((L,), jnp.float32)

    @pl.loop(0, ZB)
    def _fillz(i):
        zb16[i, :] = jnp.zeros((L,), jnp.float32)

    z0r = s * ZROWS_T

    @pl.loop(0, ZROWS_T // ZB)
    def _zero(i):
        pltpu.sync_copy(zb16, acc16.at[pl.ds(z0r + i * ZB, ZB)])

    _ztail = ZROWS_T - (ZROWS_T // ZB) * ZB
    pltpu.sync_copy(zb16.at[pl.ds(0, _ztail)],
                    acc16.at[pl.ds(z0r + (ZROWS_T // ZB) * ZB, _ztail)])
    plsc.subcore_barrier()

    lo = c * HALF
    garb = HALF + s

    @pl.loop(0, NB)
    def _sweep(t):
        cbase = s * NCH_E + t * BC
        pltpu.sync_copy(dst_hbm.at[pl.ds(cbase, BC)], dstb)
        for k in range(BC):
            for j in range(EC // L):
                d = dstb[k, pl.ds(j * L, L)]
                inh = (d >= lo) & (d < lo + HALF)
                idxb[k, pl.ds(j * L, L)] = jnp.where(inh, d - lo, garb)
            pltpu.sync_copy(ones_v, acc16.at[idxb.at[k]], add=True)
        pltpu.sync_copy(idxb, idxl_hbm.at[c, pl.ds(cbase, BC)])

    plsc.subcore_barrier()

    rbase = s * ROWS_T
    gbase = c * HALF + s * ROWS_T
    pltpu.sync_copy(acc16.at[pl.ds(rbase, ROWS_T)],
                    deg16_hbm.at[pl.ds(gbase, ROWS_T)])


def _layer_body(src_hbm, idxl_hbm, z_hbm, dinvrow_hbm, tot_hbm,
                z_out, tot_out,
                acc, zb, srcb, idxb, rows, abuf, dbuf, tbuf, sem):
    c = lax.axis_index("c")
    s = lax.axis_index("s")

    @pl.loop(0, ZB)
    def _fill(i):
        for j in range(D // L):
            zb[i, pl.ds(j * L, L)] = jnp.zeros((L,), jnp.float32)

    z0r = s * ZROWS_T

    @pl.loop(0, ZROWS_T // ZB)
    def _zero(i):
        pltpu.sync_copy(zb, acc.at[pl.ds(z0r + i * ZB, ZB)])

    _ztail = ZROWS_T - (ZROWS_T // ZB) * ZB
    pltpu.sync_copy(zb.at[pl.ds(0, _ztail)],
                    acc.at[pl.ds(z0r + (ZROWS_T // ZB) * ZB, _ztail)])
    plsc.subcore_barrier()

    @pl.loop(0, NB)
    def _sweep(t):
        cbase = s * NCH_E + t * BC
        pltpu.sync_copy(src_hbm.at[pl.ds(cbase, BC)], srcb)
        pltpu.sync_copy(idxl_hbm.at[c, pl.ds(cbase, BC)], idxb)
        cps = [None] * BC
        for k in range(KQ):
            cps[k] = pltpu.async_copy(
                z_hbm.at[srcb.at[k]], rows.at[k % KQ], sem)
        for k in range(BC):
            cps[k].wait()
            pltpu.sync_copy(rows.at[k % KQ], acc.at[idxb.at[k]], add=True)
            nk = k + KQ
            if nk < BC:
                cps[nk] = pltpu.async_copy(
                    z_hbm.at[srcb.at[nk]], rows.at[nk % KQ], sem)

    plsc.subcore_barrier()

    rbase = s * ROWS_T
    gbase = c * HALF + s * ROWS_T

    @pl.loop(0, ROWS_T // RC)
    def _out(k):
        g0 = gbase + k * RC
        pltpu.sync_copy(acc.at[pl.ds(rbase + k * RC, RC)], abuf)
        pltpu.sync_copy(dinvrow_hbm.at[pl.ds(g0, RC)], dbuf)
        pltpu.sync_copy(tot_hbm.at[pl.ds(g0, RC)], tbuf)

        @pl.loop(0, RC)
        def _scale(r):
            for j in range(D // L):
                dr = dbuf[r, pl.ds(j * L, L)]
                emb = abuf[r, pl.ds(j * L, L)] * dr
                tbuf[r, pl.ds(j * L, L)] = tbuf[r, pl.ds(j * L, L)] + emb
                abuf[r, pl.ds(j * L, L)] = emb * dr

        pltpu.sync_copy(tbuf, tot_out.at[pl.ds(g0, RC)])
        pltpu.sync_copy(abuf, z_out.at[pl.ds(g0, RC)])


def _dinv_tc_body(deg16_ref, emb0_ref, z0_ref, dinvrow_ref):
    deg = deg16_ref[:, 0:1]
    dinv = jnp.where(deg > 0.0, lax.rsqrt(jnp.maximum(deg, 1e-12)), 0.0)
    dinvrow_ref[...] = jnp.broadcast_to(dinv, (BLK, D))
    z0_ref[...] = emb0_ref[...] * dinv


@functools.lru_cache(maxsize=None)
def _build_kernels():
    mesh = plsc.VectorSubcoreMesh(
        core_axis_name="c", subcore_axis_name="s",
        num_cores=NC, num_subcores=NS)
    deg_k = pl.kernel(
        _deg_body,
        out_type=(
            jax.ShapeDtypeStruct((NPAD, L), jnp.float32),
            jax.ShapeDtypeStruct((NC, NCH_ALL, EC), jnp.int32),
        ),
        mesh=mesh,
        compiler_params=pltpu.CompilerParams(use_tc_tiling_on_sc=False),
        scratch_types=(
            pltpu.MemorySpace.VMEM_SHARED((ACC_ROWS, L), jnp.float32),
            pltpu.VMEM((EC, L), jnp.float32),
            pltpu.VMEM((ZB, L), jnp.float32),
            pltpu.VMEM((BC, EC), jnp.int32),
            pltpu.VMEM((BC, EC), jnp.int32),
        ),
    )
    layer_k = pl.kernel(
        _layer_body,
        out_type=(
            jax.ShapeDtypeStruct((NPAD, D), jnp.float32),
            jax.ShapeDtypeStruct((NPAD, D), jnp.float32),
        ),
        mesh=mesh,
        compiler_params=pltpu.CompilerParams(use_tc_tiling_on_sc=False),
        scratch_types=(
            pltpu.MemorySpace.VMEM_SHARED((ACC_ROWS, D), jnp.float32),
            pltpu.VMEM((ZB, D), jnp.float32),
            pltpu.VMEM((BC, EC), jnp.int32),
            pltpu.VMEM((BC, EC), jnp.int32),
            pltpu.VMEM((KQ, EC, D), jnp.float32),
            pltpu.VMEM((RC, D), jnp.float32),
            pltpu.VMEM((RC, D), jnp.float32),
            pltpu.VMEM((RC, D), jnp.float32),
            pltpu.SemaphoreType.DMA,
        ),
    )
    grid = (NPAD // BLK,)
    blk2 = pl.BlockSpec((BLK, D), lambda i: (i, 0))
    blk16 = pl.BlockSpec((BLK, L), lambda i: (i, 0))
    dinv_k = pl.pallas_call(
        _dinv_tc_body,
        grid=grid,
        in_specs=[blk16, blk2],
        out_specs=[blk2, blk2],
        out_shape=(
            jax.ShapeDtypeStruct((NPAD, D), jnp.float32),
            jax.ShapeDtypeStruct((NPAD, D), jnp.float32),
        ),
    )
    return deg_k, layer_k, dinv_k


def kernel(edge_index, user_weight, item_weight):
    src = edge_index[0].astype(jnp.int32)
    dst = edge_index[1].astype(jnp.int32)
    pe = EPAD - E
    pad_src = (jnp.arange(pe, dtype=jnp.int32) * 997) % N
    pad_dst = jnp.full((pe,), NPAD, dtype=jnp.int32)
    srcp = jnp.concatenate([src, pad_src]).reshape(NCH_ALL, EC)
    dstp = jnp.concatenate([dst, pad_dst]).reshape(NCH_ALL, EC)

    emb0 = jnp.concatenate([user_weight, item_weight], axis=0)
    emb0p = jnp.pad(emb0, ((0, NPAD - N), (0, 0)))

    deg_k, layer_k, dinv_k = _build_kernels()
    deg16, idxl = deg_k(dstp)
    z, dinvrow = dinv_k(deg16, emb0p)
    tot = emb0p
    for _ in range(LAYERS):
        z, tot = layer_k(srcp, idxl, z, dinvrow, tot)

    out = tot * (1.0 / ((LAYERS + 1) * (LAYERS + 1)))
    return out[:N_USERS], out[N_USERS:N]
